# baseline (speedup 1.0000x reference)
import jax
import jax.numpy as jnp
from jax.experimental import pallas as pl
from jax.experimental.pallas import tpu as pltpu

X_LEN = 50
D = 32
F = 26
FL = F * X_LEN
B = 1024
BB = 8


def _body(x_ref, t_ref, g_ref, b_ref, o_ref):
    x = x_ref[...]
    t = t_ref[...]
    w = x + t[None, :, :]
    mean = jnp.mean(w, axis=1, keepdims=True)
    var = jnp.mean(w * w, axis=1, keepdims=True) - mean * mean
    rs = jax.lax.rsqrt(var + 1e-5)
    g = g_ref[...].reshape(1, D, 1)
    bta = b_ref[...].reshape(1, D, 1)
    y = (w - mean) * (rs * g) + bta
    o_ref[...] = jnp.transpose(y, (0, 2, 1))


def kernel(x, table, gamma, beta, batch_size):
    batch = x.shape[0]
    resid = (jnp.asarray(batch_size, jnp.int32) - batch).astype(jnp.float32)
    beta_eff = beta + resid
    x3 = x.reshape(batch, D, FL)
    tfl = jnp.tile(table.T[:, None, :], (1, F, 1)).reshape(D, FL)
    out = pl.pallas_call(
        _body,
        grid=(batch // BB,),
        in_specs=[
            pl.BlockSpec((BB, D, FL), lambda i: (i, 0, 0)),
            pl.BlockSpec((D, FL), lambda i: (0, 0)),
            pl.BlockSpec((D,), lambda i: (0,)),
            pl.BlockSpec((D,), lambda i: (0,)),
        ],
        out_specs=pl.BlockSpec((BB, FL, D), lambda i: (i, 0, 0)),
        out_shape=jax.ShapeDtypeStruct((batch, FL, D), jnp.float32),
    )(x3, tfl, gamma, beta_eff)
    return out.reshape(batch, F, X_LEN, D)
